# baseline (speedup 1.0000x reference)
import functools

import jax
import jax.numpy as jnp
from jax import lax
from jax.experimental import pallas as pl
from jax.experimental.pallas import tpu as pltpu
from jax.experimental.pallas import tpu_sc as plsc

_NUM_CAPS = 1_000_000
_D = 64
_B = 16384
_L = 16
_NW = 32
_BPW = _B // _NW
_PER_W = _BPW * _D
_CH = 128
_NCH = _PER_W // _CH
_GRP = 8

_GDN = lax.GatherDimensionNumbers(
    offset_dims=(), collapsed_slice_dims=(0,), start_index_map=(0,))


def _bcast_lane(vec, l):
    idx = jnp.full((_L, 1), l, jnp.int32)
    return lax.gather(vec, idx, _GDN, slice_sizes=(1,),
                      mode=lax.GatherScatterMode.PROMISE_IN_BOUNDS)


@functools.partial(
    pl.kernel,
    mesh=plsc.VectorSubcoreMesh(core_axis_name="c", subcore_axis_name="s"),
    out_type=jax.ShapeDtypeStruct((_B * _D,), jnp.float32),
    scratch_types=[
        pltpu.VMEM((_BPW,), jnp.int32),
        pltpu.VMEM((_PER_W,), jnp.int32),
        pltpu.VMEM((_PER_W,), jnp.float32),
        pltpu.SemaphoreType.DMA,
    ],
)
def _lookup(idx_hbm, w_hbm, out_hbm, xk_v, gidx_v, vals_v, sem):
    wid = lax.axis_index("s") * 2 + lax.axis_index("c")
    base = wid * _BPW
    pltpu.sync_copy(idx_hbm.at[pl.ds(base, _BPW)], xk_v)

    dvecs = [
        (lax.iota(jnp.int32, _L) + _L * p) * _NUM_CAPS for p in range(4)
    ]

    def build(g, carry):
        xg = xk_v[pl.ds(g * _L, _L)]
        for l in range(_L):
            xb = _bcast_lane(xg, l)
            b = g * _L + l
            for p in range(4):
                gidx_v[pl.ds(b * _D + p * _L, _L)] = xb + dvecs[p]
        return carry

    lax.fori_loop(0, _BPW // _L, build, 0)

    def gather_group(g, carry):
        cps = []
        for j in range(_GRP):
            k = g * _GRP + j
            cps.append(
                pltpu.async_copy(
                    w_hbm.at[gidx_v.at[pl.ds(k * _CH, _CH)]],
                    vals_v.at[pl.ds(k * _CH, _CH)],
                    sem,
                )
            )
        for cp in cps:
            cp.wait()
        return carry

    lax.fori_loop(0, _NCH // _GRP, gather_group, 0)

    pltpu.sync_copy(vals_v, out_hbm.at[pl.ds(base * _D, _PER_W)])


def kernel(x, do_vec, W):
    idx = x.astype(jnp.int32) + jnp.asarray(do_vec).astype(jnp.int32)
    out_flat = _lookup(idx, W.reshape(-1))
    return out_flat.reshape(_B, 1, _D)


# speedup vs baseline: 1.0062x; 1.0062x over previous
import functools

import jax
import jax.numpy as jnp
from jax import lax
from jax.experimental import pallas as pl
from jax.experimental.pallas import tpu as pltpu
from jax.experimental.pallas import tpu_sc as plsc

_NUM_CAPS = 1_000_000
_D = 64
_B = 16384
_NW = 32
_DPW = _D // _NW
_CH = 4096
_NCH = _B // _CH


@functools.partial(
    pl.kernel,
    mesh=plsc.VectorSubcoreMesh(core_axis_name="c", subcore_axis_name="s"),
    out_type=jax.ShapeDtypeStruct((_D * _B,), jnp.float32),
    scratch_types=[
        pltpu.VMEM((_B,), jnp.int32),
        pltpu.VMEM((_DPW * _B,), jnp.float32),
        pltpu.SemaphoreType.DMA,
        pltpu.SemaphoreType.DMA,
    ],
)
def _lookup(idx_hbm, w_hbm, out_hbm, xk_v, vals_v, sem_x, sem_g):
    wid = lax.axis_index("s") * 2 + lax.axis_index("c")
    pltpu.sync_copy(idx_hbm, xk_v)

    for dl in range(_DPW):
        d = wid * _DPW + dl
        for c in range(_NCH):
            pltpu.async_copy(
                w_hbm.at[pl.ds(d * _NUM_CAPS, _NUM_CAPS)].at[
                    xk_v.at[pl.ds(c * _CH, _CH)]],
                vals_v.at[pl.ds(dl * _B + c * _CH, _CH)],
                sem_g,
            )

    pltpu.make_async_copy(w_hbm.at[pl.ds(0, _DPW * _B)], vals_v, sem_g).wait()
    pltpu.sync_copy(vals_v, out_hbm.at[pl.ds(wid * _DPW * _B, _DPW * _B)])


def kernel(x, do_vec, W):
    idx = x.astype(jnp.int32) + jnp.asarray(do_vec).astype(jnp.int32)
    out_t = _lookup(idx, W.reshape(-1))
    return out_t.reshape(_D, _B).T[:, None, :]


# speedup vs baseline: 10.3876x; 10.3239x over previous
import functools

import jax
import jax.numpy as jnp
from jax import lax
from jax.experimental import pallas as pl
from jax.experimental.pallas import tpu as pltpu
from jax.experimental.pallas import tpu_sc as plsc

_NUM_CAPS = 1_000_000
_D = 64
_B = 16384
_L = 16
_NW = 32
_BPW = _B // _NW
_CH = 128
_NCH = _BPW // _CH
_KB = 8
_STEPS = -(-_NUM_CAPS // (256 * _KB))
_ROWS = _STEPS * _KB * 128


def _transpose_body(w_ref, out_ref):
    for k in range(_KB):
        a = w_ref[:, pl.ds(k * 256, 128)]
        b = w_ref[:, pl.ds(k * 256 + 128, 128)]
        e = lax.transpose(a, (1, 0))
        o = lax.transpose(b, (1, 0))
        out_ref[pl.ds(k * 128, 128), :] = lax.concatenate([e, o], 1)


_transpose = pl.pallas_call(
    _transpose_body,
    grid=(_STEPS,),
    in_specs=[pl.BlockSpec((_D, 256 * _KB), lambda i: (0, i))],
    out_specs=pl.BlockSpec((_KB * 128, 128), lambda i: (i, 0)),
    out_shape=jax.ShapeDtypeStruct((_ROWS, 128), jnp.float32),
)


_GDN = lax.GatherDimensionNumbers(
    offset_dims=(), collapsed_slice_dims=(0,), start_index_map=(0,))


def _bcast_lane(vec, l):
    idx = jnp.full((_L, 1), l, jnp.int32)
    return lax.gather(vec, idx, _GDN, slice_sizes=(1,),
                      mode=lax.GatherScatterMode.PROMISE_IN_BOUNDS)


@functools.partial(
    pl.kernel,
    mesh=plsc.VectorSubcoreMesh(core_axis_name="c", subcore_axis_name="s"),
    out_type=jax.ShapeDtypeStruct((_B, _D), jnp.float32),
    compiler_params=pltpu.CompilerParams(use_tc_tiling_on_sc=False),
    scratch_types=[
        pltpu.VMEM((_BPW,), jnp.int32),
        pltpu.VMEM((_BPW,), jnp.int32),
        pltpu.VMEM((_BPW, 2 * _D), jnp.float32),
        pltpu.VMEM((_BPW, _D), jnp.float32),
        pltpu.SemaphoreType.DMA,
    ],
)
def _gather_rows(idx_hbm, wt_hbm, out_hbm, xk_v, xh_v, rows_v, obuf_v, sem):
    wid = lax.axis_index("s") * 2 + lax.axis_index("c")
    base = wid * _BPW
    pltpu.sync_copy(idx_hbm.at[pl.ds(base, _BPW)], xk_v)

    def build(g, carry):
        xg = xk_v[pl.ds(g * _L, _L)]
        xh_v[pl.ds(g * _L, _L)] = lax.shift_left(
            lax.shift_right_logical(xg, 8), 7) | (xg & 127)
        return carry

    lax.fori_loop(0, _BPW // _L, build, 0)

    for k in range(_NCH):
        pltpu.async_copy(
            wt_hbm.at[xh_v.at[pl.ds(k * _CH, _CH)]],
            rows_v.at[pl.ds(k * _CH, _CH), :],
            sem,
        )
    pltpu.make_async_copy(
        wt_hbm.at[pl.ds(0, _BPW)], rows_v, sem).wait()

    def sel(g, carry):
        xg = xk_v[pl.ds(g * _L, _L)]
        for l in range(_L):
            m = 0 - (lax.shift_right_logical(_bcast_lane(xg, l), 7) & 1)
            b = g * _L + l
            for j in range(_D // _L):
                lo = lax.bitcast_convert_type(
                    rows_v[b, pl.ds(j * _L, _L)], jnp.int32)
                hi = lax.bitcast_convert_type(
                    rows_v[b, pl.ds(_D + j * _L, _L)], jnp.int32)
                obuf_v[b, pl.ds(j * _L, _L)] = lax.bitcast_convert_type(
                    (hi & m) | (lo & (m ^ -1)), jnp.float32)
        return carry

    lax.fori_loop(0, _BPW // _L, sel, 0)

    pltpu.sync_copy(obuf_v, out_hbm.at[pl.ds(base, _BPW)])


def kernel(x, do_vec, W):
    idx = x.astype(jnp.int32) + jnp.asarray(do_vec).astype(jnp.int32)
    wt = _transpose(W)
    return _gather_rows(idx, wt).reshape(_B, 1, _D)


# speedup vs baseline: 16.9067x; 1.6276x over previous
import functools

import jax
import jax.numpy as jnp
from jax import lax
from jax.experimental import pallas as pl
from jax.experimental.pallas import tpu as pltpu
from jax.experimental.pallas import tpu_sc as plsc

_NUM_CAPS = 1_000_000
_D = 64
_B = 16384
_L = 16
_NW = 32
_BPW = _B // _NW
_CH = 128
_NCH = _BPW // _CH
_KB = 32
_STEPS = -(-_NUM_CAPS // (256 * _KB))
_ROWS = _STEPS * _KB * 128


def _transpose_body(w_ref, out_ref):
    for k in range(_KB):
        a = w_ref[:, pl.ds(k * 256, 128)]
        b = w_ref[:, pl.ds(k * 256 + 128, 128)]
        e = lax.transpose(a, (1, 0))
        o = lax.transpose(b, (1, 0))
        out_ref[pl.ds(k * 128, 128), :] = lax.concatenate([e, o], 1)


_transpose = pl.pallas_call(
    _transpose_body,
    grid=(_STEPS,),
    in_specs=[pl.BlockSpec((_D, 256 * _KB), lambda i: (0, i))],
    out_specs=pl.BlockSpec((_KB * 128, 128), lambda i: (i, 0)),
    out_shape=jax.ShapeDtypeStruct((_ROWS, 128), jnp.float32),
)


_GDN = lax.GatherDimensionNumbers(
    offset_dims=(), collapsed_slice_dims=(0,), start_index_map=(0,))


def _bcast_lane(vec, l):
    idx = jnp.full((_L, 1), l, jnp.int32)
    return lax.gather(vec, idx, _GDN, slice_sizes=(1,),
                      mode=lax.GatherScatterMode.PROMISE_IN_BOUNDS)


@functools.partial(
    pl.kernel,
    mesh=plsc.VectorSubcoreMesh(core_axis_name="c", subcore_axis_name="s"),
    out_type=jax.ShapeDtypeStruct((_B, _D), jnp.float32),
    compiler_params=pltpu.CompilerParams(use_tc_tiling_on_sc=False),
    scratch_types=[
        pltpu.VMEM((_BPW,), jnp.int32),
        pltpu.VMEM((_BPW,), jnp.int32),
        pltpu.VMEM((_BPW, 2 * _D), jnp.float32),
        pltpu.VMEM((_BPW, _D), jnp.float32),
        pltpu.SemaphoreType.DMA,
    ],
)
def _gather_rows(idx_hbm, wt_hbm, out_hbm, xk_v, xh_v, rows_v, obuf_v, sem):
    wid = lax.axis_index("s") * 2 + lax.axis_index("c")
    base = wid * _BPW
    pltpu.sync_copy(idx_hbm.at[pl.ds(base, _BPW)], xk_v)

    def build(g, carry):
        xg = xk_v[pl.ds(g * _L, _L)]
        xh_v[pl.ds(g * _L, _L)] = lax.shift_left(
            lax.shift_right_logical(xg, 8), 7) | (xg & 127)
        return carry

    lax.fori_loop(0, _BPW // _L, build, 0)

    for k in range(_NCH):
        pltpu.async_copy(
            wt_hbm.at[xh_v.at[pl.ds(k * _CH, _CH)]],
            rows_v.at[pl.ds(k * _CH, _CH), :],
            sem,
        )
    pltpu.make_async_copy(
        wt_hbm.at[pl.ds(0, _BPW)], rows_v, sem).wait()

    def sel(g, carry):
        xg = xk_v[pl.ds(g * _L, _L)]
        for l in range(_L):
            m = 0 - (lax.shift_right_logical(_bcast_lane(xg, l), 7) & 1)
            b = g * _L + l
            for j in range(_D // _L):
                lo = lax.bitcast_convert_type(
                    rows_v[b, pl.ds(j * _L, _L)], jnp.int32)
                hi = lax.bitcast_convert_type(
                    rows_v[b, pl.ds(_D + j * _L, _L)], jnp.int32)
                obuf_v[b, pl.ds(j * _L, _L)] = lax.bitcast_convert_type(
                    (hi & m) | (lo & (m ^ -1)), jnp.float32)
        return carry

    lax.fori_loop(0, _BPW // _L, sel, 0)

    pltpu.sync_copy(obuf_v, out_hbm.at[pl.ds(base, _BPW)])


def kernel(x, do_vec, W):
    idx = x.astype(jnp.int32) + jnp.asarray(do_vec).astype(jnp.int32)
    wt = _transpose(W)
    return _gather_rows(idx, wt).reshape(_B, 1, _D)


# speedup vs baseline: 19.9865x; 1.1822x over previous
import functools

import jax
import jax.numpy as jnp
from jax import lax
from jax.experimental import pallas as pl
from jax.experimental.pallas import tpu as pltpu
from jax.experimental.pallas import tpu_sc as plsc

_NUM_CAPS = 1_000_000
_D = 64
_B = 16384
_L = 16
_NW = 32
_BPW = _B // _NW
_CH = 128
_NCH = _BPW // _CH
_KB = 128
_STEPS = -(-_NUM_CAPS // (256 * _KB))
_ROWS = _STEPS * _KB * 128


def _transpose_body(w_ref, out_ref):
    for k in range(_KB):
        a = w_ref[:, pl.ds(k * 256, 128)]
        b = w_ref[:, pl.ds(k * 256 + 128, 128)]
        e = lax.transpose(a, (1, 0))
        o = lax.transpose(b, (1, 0))
        out_ref[pl.ds(k * 128, 128), :] = lax.concatenate([e, o], 1)


_transpose = pl.pallas_call(
    _transpose_body,
    grid=(_STEPS,),
    in_specs=[pl.BlockSpec((_D, 256 * _KB), lambda i: (0, i))],
    out_specs=pl.BlockSpec((_KB * 128, 128), lambda i: (i, 0)),
    out_shape=jax.ShapeDtypeStruct((_ROWS, 128), jnp.float32),
)


_GDN = lax.GatherDimensionNumbers(
    offset_dims=(), collapsed_slice_dims=(0,), start_index_map=(0,))


def _bcast_lane(vec, l):
    idx = jnp.full((_L, 1), l, jnp.int32)
    return lax.gather(vec, idx, _GDN, slice_sizes=(1,),
                      mode=lax.GatherScatterMode.PROMISE_IN_BOUNDS)


@functools.partial(
    pl.kernel,
    mesh=plsc.VectorSubcoreMesh(core_axis_name="c", subcore_axis_name="s"),
    out_type=jax.ShapeDtypeStruct((_B, _D), jnp.float32),
    compiler_params=pltpu.CompilerParams(use_tc_tiling_on_sc=False),
    scratch_types=[
        pltpu.VMEM((_BPW,), jnp.int32),
        pltpu.VMEM((_BPW,), jnp.int32),
        pltpu.VMEM((_BPW, 2 * _D), jnp.float32),
        pltpu.VMEM((_BPW, _D), jnp.float32),
        pltpu.SemaphoreType.DMA,
    ],
)
def _gather_rows(idx_hbm, wt_hbm, out_hbm, xk_v, xh_v, rows_v, obuf_v, sem):
    wid = lax.axis_index("s") * 2 + lax.axis_index("c")
    base = wid * _BPW
    pltpu.sync_copy(idx_hbm.at[pl.ds(base, _BPW)], xk_v)

    def build(g, carry):
        xg = xk_v[pl.ds(g * _L, _L)]
        xh_v[pl.ds(g * _L, _L)] = lax.shift_left(
            lax.shift_right_logical(xg, 8), 7) | (xg & 127)
        return carry

    lax.fori_loop(0, _BPW // _L, build, 0)

    for k in range(_NCH):
        pltpu.async_copy(
            wt_hbm.at[xh_v.at[pl.ds(k * _CH, _CH)]],
            rows_v.at[pl.ds(k * _CH, _CH), :],
            sem,
        )
    pltpu.make_async_copy(
        wt_hbm.at[pl.ds(0, _BPW)], rows_v, sem).wait()

    def sel(g, carry):
        xg = xk_v[pl.ds(g * _L, _L)]
        for l in range(_L):
            m = 0 - (lax.shift_right_logical(_bcast_lane(xg, l), 7) & 1)
            b = g * _L + l
            for j in range(_D // _L):
                lo = lax.bitcast_convert_type(
                    rows_v[b, pl.ds(j * _L, _L)], jnp.int32)
                hi = lax.bitcast_convert_type(
                    rows_v[b, pl.ds(_D + j * _L, _L)], jnp.int32)
                obuf_v[b, pl.ds(j * _L, _L)] = lax.bitcast_convert_type(
                    (hi & m) | (lo & (m ^ -1)), jnp.float32)
        return carry

    lax.fori_loop(0, _BPW // _L, sel, 0)

    pltpu.sync_copy(obuf_v, out_hbm.at[pl.ds(base, _BPW)])


def kernel(x, do_vec, W):
    idx = x.astype(jnp.int32) + jnp.asarray(do_vec).astype(jnp.int32)
    wt = _transpose(W)
    return _gather_rows(idx, wt).reshape(_B, 1, _D)
